# Initial kernel scaffold; baseline (speedup 1.0000x reference)
import jax
import jax.numpy as jnp
from jax.experimental import pallas as pl


def kernel(token_ids, tok_table, pos_table):
    raise NotImplementedError("write your pallas kernel here")


# speedup vs baseline: 1.0194x; 1.0194x over previous
import functools

import jax
import jax.numpy as jnp
from jax import lax
from jax.experimental import pallas as pl
from jax.experimental.pallas import tpu as pltpu
from jax.experimental.pallas import tpu_sc as plsc

NW = 32
CHUNK = 128
LANES = 16


def _emb_kernel(ids_hbm, tok_hbm, pos_hbm, out_hbm, idx_v, rows_v, pos_v,
                gsem):
    n_chunks = ids_hbm.shape[1]
    seq = pos_hbm.shape[0]
    rows_per_w = n_chunks * CHUNK

    wid = lax.axis_index("s") * 2 + lax.axis_index("c")
    base = wid * rows_per_w
    s0 = lax.rem(base, seq)

    pltpu.sync_copy(ids_hbm.at[wid], idx_v)

    for c in range(n_chunks):
        gather = pltpu.async_copy(tok_hbm.at[idx_v.at[c]], rows_v, gsem)
        pltpu.sync_copy(pos_hbm.at[pl.ds(s0 + c * CHUNK, CHUNK)], pos_v)
        gather.wait()

        def add_body(i, _):
            for j in range(0, 128, LANES):
                sl = pl.ds(j, LANES)
                rows_v[i, sl] = rows_v[i, sl] + pos_v[i, sl]
            return 0

        lax.fori_loop(0, CHUNK, add_body, 0)
        pltpu.sync_copy(rows_v, out_hbm.at[pl.ds(base + c * CHUNK, CHUNK)])


@functools.partial(jax.jit, static_argnums=())
def kernel(token_ids, tok_table, pos_table):
    b, s = token_ids.shape
    d = tok_table.shape[1]
    total = b * s
    assert total % (NW * CHUNK) == 0 and d == 128
    n_chunks = total // (NW * CHUNK)

    ids = token_ids.reshape(NW, n_chunks, CHUNK).astype(jnp.int32)

    mesh = plsc.VectorSubcoreMesh(core_axis_name="c", subcore_axis_name="s")
    run = pl.kernel(
        _emb_kernel,
        out_type=jax.ShapeDtypeStruct((total, d), jnp.float32),
        mesh=mesh,
        scratch_types=[
            pltpu.VMEM((n_chunks, CHUNK), jnp.int32),
            pltpu.VMEM((CHUNK, 128), jnp.float32),
            pltpu.VMEM((CHUNK, 128), jnp.float32),
            pltpu.SemaphoreType.DMA,
        ],
    )
    out = run(ids, tok_table, pos_table)
    return out.reshape(b, s, d)


# speedup vs baseline: 1.3092x; 1.2843x over previous
import functools

import jax
import jax.numpy as jnp
from jax import lax
from jax.experimental import pallas as pl
from jax.experimental.pallas import tpu as pltpu
from jax.experimental.pallas import tpu_sc as plsc

NW = 32
CHUNK = 128
LANES = 16
NBR = 4
NBP = 3


def _emb_kernel(ids_hbm, tok_hbm, pos_hbm, out_hbm, idx_v, rows_v, pos_v,
                gsem, psem, osem):
    n_chunks = ids_hbm.shape[1]
    seq = pos_hbm.shape[0]
    rows_per_w = n_chunks * CHUNK

    wid = lax.axis_index("s") * 2 + lax.axis_index("c")
    base = wid * rows_per_w
    s0 = lax.rem(base, seq)

    pltpu.sync_copy(ids_hbm.at[wid], idx_v)

    g = [None] * n_chunks
    p = [None] * n_chunks
    o = [None] * n_chunks

    def issue(c):
        g[c] = pltpu.async_copy(tok_hbm.at[idx_v.at[c]], rows_v.at[c % NBR],
                                gsem)
        p[c] = pltpu.async_copy(pos_hbm.at[pl.ds(s0 + c * CHUNK, CHUNK)],
                                pos_v.at[c % NBP], psem)

    for c in range(min(NBR - 1, n_chunks)):
        issue(c)

    for c in range(n_chunks):
        g[c].wait()
        p[c].wait()
        rv = rows_v.at[c % NBR]
        pv = pos_v.at[c % NBP]

        @plsc.parallel_loop(0, CHUNK, step=1, unroll=2)
        def add_body(i):
            for j in range(0, 128, LANES):
                sl = pl.ds(j, LANES)
                rv[i, sl] = rv[i, sl] + pv[i, sl]

        o[c] = pltpu.async_copy(rv, out_hbm.at[pl.ds(base + c * CHUNK, CHUNK)],
                                osem)
        nc = c + NBR - 1
        if nc < n_chunks:
            if c >= 1:
                o[c - 1].wait()
            issue(nc)

    for c in range(max(0, n_chunks - NBR), n_chunks):
        o[c].wait()


@functools.partial(jax.jit, static_argnums=())
def kernel(token_ids, tok_table, pos_table):
    b, s = token_ids.shape
    d = tok_table.shape[1]
    total = b * s
    assert total % (NW * CHUNK) == 0 and d == 128
    n_chunks = total // (NW * CHUNK)

    ids = token_ids.reshape(NW, n_chunks, CHUNK).astype(jnp.int32)

    mesh = plsc.VectorSubcoreMesh(core_axis_name="c", subcore_axis_name="s")
    run = pl.kernel(
        _emb_kernel,
        out_type=jax.ShapeDtypeStruct((total, d), jnp.float32),
        mesh=mesh,
        scratch_types=[
            pltpu.VMEM((n_chunks, CHUNK), jnp.int32),
            pltpu.VMEM((NBR, CHUNK, 128), jnp.float32),
            pltpu.VMEM((NBP, CHUNK, 128), jnp.float32),
            pltpu.SemaphoreType.DMA,
            pltpu.SemaphoreType.DMA,
            pltpu.SemaphoreType.DMA,
        ],
    )
    out = run(ids, tok_table, pos_table)
    return out.reshape(b, s, d)


# speedup vs baseline: 1.5212x; 1.1619x over previous
import functools

import jax
import jax.numpy as jnp
from jax import lax
from jax.experimental import pallas as pl
from jax.experimental.pallas import tpu as pltpu
from jax.experimental.pallas import tpu_sc as plsc

NW = 32
CHUNK = 128
LANES = 16
NBR = 4


def _emb_kernel(ids_hbm, tok_hbm, pos_hbm, out_hbm, idx_v, rows_v, pos_v,
                gsem, psem, osem):
    batch, seq = ids_hbm.shape
    pos_per_w = seq // NW
    n_pos_chunks = pos_per_w // CHUNK
    n_units = n_pos_chunks * batch

    wid = lax.axis_index("s") * 2 + lax.axis_index("c")
    col0 = wid * pos_per_w

    pltpu.sync_copy(ids_hbm.at[:, pl.ds(col0, pos_per_w)], idx_v)

    units = [(c, b) for c in range(n_pos_chunks) for b in range(batch)]
    g = [None] * n_units
    p = [None] * n_pos_chunks
    o = [None] * n_units

    for c in range(n_pos_chunks):
        p[c] = pltpu.async_copy(pos_hbm.at[pl.ds(col0 + c * CHUNK, CHUNK)],
                                pos_v.at[c % 2], psem)

    def issue(u):
        c, b = units[u]
        idx = idx_v.at[b, pl.ds(c * CHUNK, CHUNK)]
        g[u] = pltpu.async_copy(tok_hbm.at[idx], rows_v.at[u % NBR], gsem)

    for u in range(min(NBR - 1, n_units)):
        issue(u)

    for u, (c, b) in enumerate(units):
        g[u].wait()
        if b == 0:
            p[c].wait()
        rv = rows_v.at[u % NBR]
        pv = pos_v.at[c % 2]

        @plsc.parallel_loop(0, CHUNK, step=1, unroll=2)
        def add_body(i):
            for j in range(0, 128, LANES):
                sl = pl.ds(j, LANES)
                rv[i, sl] = rv[i, sl] + pv[i, sl]

        o[u] = pltpu.async_copy(
            rv, out_hbm.at[b, pl.ds(col0 + c * CHUNK, CHUNK)], osem)
        nu = u + NBR - 1
        if nu < n_units:
            if u >= 1:
                o[u - 1].wait()
            issue(nu)

    for u in range(max(0, n_units - NBR), n_units):
        o[u].wait()


@functools.partial(jax.jit, static_argnums=())
def kernel(token_ids, tok_table, pos_table):
    b, s = token_ids.shape
    d = tok_table.shape[1]
    assert s % (NW * CHUNK) == 0 and d == 128

    ids = token_ids.astype(jnp.int32)
    pos_per_w = s // NW

    mesh = plsc.VectorSubcoreMesh(core_axis_name="c", subcore_axis_name="s")
    run = pl.kernel(
        _emb_kernel,
        out_type=jax.ShapeDtypeStruct((b, s, d), jnp.float32),
        mesh=mesh,
        scratch_types=[
            pltpu.VMEM((b, pos_per_w), jnp.int32),
            pltpu.VMEM((NBR, CHUNK, 128), jnp.float32),
            pltpu.VMEM((2, CHUNK, 128), jnp.float32),
            pltpu.SemaphoreType.DMA,
            pltpu.SemaphoreType.DMA,
            pltpu.SemaphoreType.DMA,
        ],
    )
    return run(ids, tok_table, pos_table)
